# baseline (speedup 1.0000x reference)
import jax
import jax.numpy as jnp
from jax import lax
from jax.experimental import pallas as pl

N_POINTS = 2000000
NUM_PILLARS = 100000
MAX_NODES = 32
NUM_BINS = 8
IN_CH = 5
OUT_CH = 128

BLK = 1000


def _head_body(d_ref, ez_ref, rex_ref, sc_ref, lw_ref, emb_ref, aw_ref,
               smat_ref, rm_ref, rv_ref, gm_ref, bt_ref, y_ref, attn_ref):
    x2 = d_ref[...]
    z2 = jnp.dot(x2, ez_ref[...], preferred_element_type=jnp.float32, precision=lax.Precision.HIGHEST)
    sums_parts = []
    cnt_parts = []
    for b in range(NUM_BINS):
        lo = -5.0 + float(b)
        hi = lo + 1.0
        mb = ((z2 > lo) & (z2 < hi)).astype(jnp.float32)
        mex = jnp.dot(mb, rex_ref[...], preferred_element_type=jnp.float32, precision=lax.Precision.HIGHEST)
        prod = x2 * mex
        s8 = jnp.dot(prod, sc_ref[...], preferred_element_type=jnp.float32, precision=lax.Precision.HIGHEST)
        sums_parts.append(s8[:, :IN_CH])
        cnt = jnp.sum(mb, axis=1, keepdims=True)
        cnt_parts.append(jnp.broadcast_to(cnt, (BLK, IN_CH)))
    sums = jnp.concatenate(sums_parts, axis=1)
    cnt_rep = jnp.concatenate(cnt_parts, axis=1)
    mean = sums / jnp.maximum(cnt_rep, 1.0)
    xf = jnp.where(cnt_rep > 0.0, mean, emb_ref[...])
    attn_ref[...] = jnp.dot(xf * aw_ref[...], smat_ref[...],
                            preferred_element_type=jnp.float32, precision=lax.Precision.HIGHEST)
    y = jnp.dot(xf, lw_ref[...], preferred_element_type=jnp.float32, precision=lax.Precision.HIGHEST)
    scale = gm_ref[...] * lax.rsqrt(rv_ref[...] + 1e-3)
    y = (y - rm_ref[...]) * scale + bt_ref[...]
    y_ref[...] = jnp.maximum(y, 0.0)


def kernel(x, idx, batch_dict, emb1, attn_w, lin_w, gamma, beta,
           running_mean, running_var):
    del batch_dict
    idx = idx.astype(jnp.int32)
    xs5 = x[:, 1:]

    order = jnp.argsort(idx)
    idx_s = idx[order]
    xs = xs5[order]
    counts = jnp.bincount(idx, length=NUM_PILLARS)
    starts = jnp.cumsum(counts) - counts
    pos = jnp.arange(N_POINTS, dtype=jnp.int32) - starts[idx_s].astype(jnp.int32)
    posc = jnp.where(pos < MAX_NODES, pos, MAX_NODES)
    dense = jnp.zeros((NUM_PILLARS, MAX_NODES + 1, IN_CH), dtype=x.dtype)
    dense = dense.at[idx_s, posc].set(xs)
    dense2 = dense[:, :MAX_NODES].reshape(NUM_PILLARS, MAX_NODES * IN_CH)

    s_ar = jnp.arange(MAX_NODES)
    j_ar = jnp.arange(MAX_NODES * IN_CH)
    ez = (j_ar[:, None] == s_ar[None, :] * IN_CH + 2).astype(jnp.float32)
    rex = (j_ar[None, :] // IN_CH == s_ar[:, None]).astype(jnp.float32)
    sc = (j_ar[:, None] % IN_CH == jnp.arange(8)[None, :]).astype(jnp.float32)
    rep8 = (jnp.arange(40)[None, :] // IN_CH
            == jnp.arange(8)[:, None]).astype(jnp.float32)
    smat = rep8.T
    lin_wt = lin_w.T
    emb_row = jnp.tile(emb1, (1, NUM_BINS))
    attn_row = jnp.tile(attn_w, (1, NUM_BINS))
    rm = running_mean.reshape(1, OUT_CH)
    rv = running_var.reshape(1, OUT_CH)
    gm = gamma.reshape(1, OUT_CH)
    bt = beta.reshape(1, OUT_CH)

    rowspec = lambda shape: pl.BlockSpec(shape, lambda i: (0, 0))
    y, attn = pl.pallas_call(
        _head_body,
        grid=(NUM_PILLARS // BLK,),
        in_specs=[
            pl.BlockSpec((BLK, MAX_NODES * IN_CH), lambda i: (i, 0)),
            rowspec((MAX_NODES * IN_CH, MAX_NODES)),
            rowspec((MAX_NODES, MAX_NODES * IN_CH)),
            rowspec((MAX_NODES * IN_CH, 8)),
            rowspec((40, OUT_CH)),
            rowspec((1, 40)),
            rowspec((1, 40)),
            rowspec((40, 8)),
            rowspec((1, OUT_CH)),
            rowspec((1, OUT_CH)),
            rowspec((1, OUT_CH)),
            rowspec((1, OUT_CH)),
        ],
        out_specs=[
            pl.BlockSpec((BLK, OUT_CH), lambda i: (i, 0)),
            pl.BlockSpec((BLK, 8), lambda i: (i, 0)),
        ],
        out_shape=[
            jax.ShapeDtypeStruct((NUM_PILLARS, OUT_CH), jnp.float32),
            jax.ShapeDtypeStruct((NUM_PILLARS, 8), jnp.float32),
        ],
    )(dense2, ez, rex, sc, lin_wt, emb_row, attn_row, smat, rm, rv, gm, bt)

    z_coord = jnp.broadcast_to(
        (jnp.arange(NUM_BINS, dtype=jnp.float32) - 4.5).reshape(1, NUM_BINS, 1),
        (NUM_PILLARS, NUM_BINS, 1))
    return (y, z_coord, attn)
